# baseline (speedup 1.0000x reference)
import jax
import jax.numpy as jnp
from jax.experimental import pallas as pl
from jax.experimental.pallas import tpu as pltpu


def _gat(x, addm, w_ref, as_ref, ad_ref, b_ref):
    h = jnp.dot(x, w_ref[...], preferred_element_type=jnp.float32)
    als = jax.lax.dot_general(
        h, as_ref[...], (((1,), (1,)), ((), ())),
        preferred_element_type=jnp.float32)
    ald = jax.lax.dot_general(
        ad_ref[...], h, (((1,), (1,)), ((), ())),
        preferred_element_type=jnp.float32)
    s = als + ald
    s = jnp.maximum(s, 0.2 * s)
    v = jnp.max(als) + ald
    m = jnp.maximum(v, 0.2 * v)
    ex = jnp.exp2(s + addm - m)
    hp = jnp.concatenate(
        [h, jnp.ones((h.shape[0], 1), jnp.float32)], axis=1)
    aug = jax.lax.dot_general(ex, hp, (((0,), (0,)), ((), ())),
                              preferred_element_type=jnp.float32)
    dout = h.shape[1]
    out = aug[:, :dout] / (aug[:, dout:dout + 1] + 1e-16)
    return jnp.maximum(out + b_ref[...], 0.0)


def _body(x_ref, e_ref,
          w0, as0, ad0, b0, w1, as1, ad1, b1, w2, as2, ad2, b2,
          w3, as3, ad3, b3, w4, as4, ad4, b4,
          recon_ref, xr_ref, z_ref):
    addm = jnp.where(e_ref[...] != 0, 0.0, -jnp.inf)
    h = _gat(x_ref[...], addm, w0, as0, ad0, b0)
    z = _gat(h, addm, w1, as1, ad1, b1)
    z_ref[...] = z
    re = _gat(z, addm, w2, as2, ad2, b2)
    recon_ref[...] = jax.nn.sigmoid(
        jax.lax.dot_general(re, re, (((1,), (1,)), ((), ())),
                            preferred_element_type=jnp.float32))
    xr = _gat(z, addm, w3, as3, ad3, b3)
    xr_ref[...] = _gat(xr, addm, w4, as4, ad4, b4)


def kernel(x, edge_index, W0, as0, ad0, b0, W1, as1, ad1, b1, W2, as2, ad2, b2,
           W3, as3, ad3, b3, W4, as4, ad4, b4):
    n, in_ch = x.shape
    mid = W1.shape[1]
    log2e = 1.4426950408889634
    as0, ad0 = as0 * log2e, ad0 * log2e
    as1, ad1 = as1 * log2e, ad1 * log2e
    as2, ad2 = as2 * log2e, ad2 * log2e
    as3, ad3 = as3 * log2e, ad3 * log2e
    as4, ad4 = as4 * log2e, ad4 * log2e
    vmem = pl.BlockSpec(memory_space=pltpu.MemorySpace.VMEM)
    recon, xr, z = pl.pallas_call(
        _body,
        in_specs=[vmem] * 22,
        out_specs=(vmem, vmem, vmem),
        out_shape=(
            jax.ShapeDtypeStruct((n, n), jnp.float32),
            jax.ShapeDtypeStruct((n, in_ch), jnp.float32),
            jax.ShapeDtypeStruct((n, mid), jnp.float32),
        ),
    )(x, edge_index,
      W0, as0.reshape(1, -1), ad0.reshape(1, -1), b0.reshape(1, -1),
      W1, as1.reshape(1, -1), ad1.reshape(1, -1), b1.reshape(1, -1),
      W2, as2.reshape(1, -1), ad2.reshape(1, -1), b2.reshape(1, -1),
      W3, as3.reshape(1, -1), ad3.reshape(1, -1), b3.reshape(1, -1),
      W4, as4.reshape(1, -1), ad4.reshape(1, -1), b4.reshape(1, -1))
    return recon, xr, z


# speedup vs baseline: 1.4073x; 1.4073x over previous
import jax
import jax.numpy as jnp
from jax.experimental import pallas as pl
from jax.experimental.pallas import tpu as pltpu


def _gat(x, addm, w_ref, as_ref, ad_ref, b_ref):
    h = jnp.dot(x, w_ref[...], preferred_element_type=jnp.float32)
    als = jax.lax.dot_general(
        h, as_ref[...], (((1,), (1,)), ((), ())),
        preferred_element_type=jnp.float32)
    ald = jax.lax.dot_general(
        ad_ref[...], h, (((1,), (1,)), ((), ())),
        preferred_element_type=jnp.float32)
    s = als + ald
    s = jnp.maximum(s, 0.2 * s)
    v = jnp.max(als) + ald
    m = jnp.maximum(v, 0.2 * v)
    ex = jnp.exp(s + addm - m)
    hp = jnp.concatenate(
        [h, jnp.ones((h.shape[0], 1), jnp.float32)], axis=1)
    aug = jax.lax.dot_general(
        ex.astype(jnp.bfloat16), hp.astype(jnp.bfloat16),
        (((0,), (0,)), ((), ())), preferred_element_type=jnp.float32)
    dout = h.shape[1]
    out = aug[:, :dout] / (aug[:, dout:dout + 1] + 1e-16)
    return jnp.maximum(out + b_ref[...], 0.0)


def _body(x_ref, e_ref,
          w0, as0, ad0, b0, w1, as1, ad1, b1, w2, as2, ad2, b2,
          w3, as3, ad3, b3, w4, as4, ad4, b4,
          recon_ref, xr_ref, z_ref):
    addm = jnp.where(e_ref[...] != 0, 0.0, -jnp.inf)
    h = _gat(x_ref[...], addm, w0, as0, ad0, b0)
    z = _gat(h, addm, w1, as1, ad1, b1)
    z_ref[...] = z
    re = _gat(z, addm, w2, as2, ad2, b2)
    recon_ref[...] = jax.nn.sigmoid(
        jax.lax.dot_general(re, re, (((1,), (1,)), ((), ())),
                            preferred_element_type=jnp.float32))
    xr = _gat(z, addm, w3, as3, ad3, b3)
    xr_ref[...] = _gat(xr, addm, w4, as4, ad4, b4)


def kernel(x, edge_index, W0, as0, ad0, b0, W1, as1, ad1, b1, W2, as2, ad2, b2,
           W3, as3, ad3, b3, W4, as4, ad4, b4):
    n, in_ch = x.shape
    mid = W1.shape[1]
    vmem = pl.BlockSpec(memory_space=pltpu.MemorySpace.VMEM)
    recon, xr, z = pl.pallas_call(
        _body,
        in_specs=[vmem] * 22,
        out_specs=(vmem, vmem, vmem),
        out_shape=(
            jax.ShapeDtypeStruct((n, n), jnp.float32),
            jax.ShapeDtypeStruct((n, in_ch), jnp.float32),
            jax.ShapeDtypeStruct((n, mid), jnp.float32),
        ),
    )(x, edge_index,
      W0, as0.reshape(1, -1), ad0.reshape(1, -1), b0.reshape(1, -1),
      W1, as1.reshape(1, -1), ad1.reshape(1, -1), b1.reshape(1, -1),
      W2, as2.reshape(1, -1), ad2.reshape(1, -1), b2.reshape(1, -1),
      W3, as3.reshape(1, -1), ad3.reshape(1, -1), b3.reshape(1, -1),
      W4, as4.reshape(1, -1), ad4.reshape(1, -1), b4.reshape(1, -1))
    return recon, xr, z


# speedup vs baseline: 1.4081x; 1.0005x over previous
import jax
import jax.numpy as jnp
from jax.experimental import pallas as pl
from jax.experimental.pallas import tpu as pltpu


def _gat(x, addm, w_ref, as_ref, ad_ref, b_ref):
    h = jnp.dot(x, w_ref[...], preferred_element_type=jnp.float32)
    als = jax.lax.dot_general(
        h, as_ref[...], (((1,), (1,)), ((), ())),
        preferred_element_type=jnp.float32)
    ald = jax.lax.dot_general(
        ad_ref[...], h, (((1,), (1,)), ((), ())),
        preferred_element_type=jnp.float32)
    s = als + ald
    s = jnp.maximum(s, 0.2 * s)
    v = jnp.max(als) + ald
    m = jnp.maximum(v, 0.2 * v)
    ex = jnp.exp(s + addm - m)
    hp = jnp.concatenate(
        [h, jnp.ones((h.shape[0], 1), jnp.float32)], axis=1)
    aug = jax.lax.dot_general(
        ex.astype(jnp.bfloat16), hp.astype(jnp.bfloat16),
        (((0,), (0,)), ((), ())), preferred_element_type=jnp.float32)
    dout = h.shape[1]
    out = aug[:, :dout] / (aug[:, dout:dout + 1] + 1e-16)
    return jnp.maximum(out + b_ref[...], 0.0)


def _body(x_ref, e_ref,
          w0, as0, ad0, b0, w1, as1, ad1, b1, w2, as2, ad2, b2,
          w3, as3, ad3, b3, w4, as4, ad4, b4,
          recon_ref, xr_ref, z_ref):
    addm = jnp.where(e_ref[...] != 0, 0.0, -jnp.inf)
    h = _gat(x_ref[...], addm, w0, as0, ad0, b0)
    z = _gat(h, addm, w1, as1, ad1, b1)
    z_ref[...] = z
    re = _gat(z, addm, w2, as2, ad2, b2)
    reb = re.astype(jnp.bfloat16)
    recon_ref[...] = jax.nn.sigmoid(
        jax.lax.dot_general(reb, reb, (((1,), (1,)), ((), ())),
                            preferred_element_type=jnp.float32))
    xr = _gat(z, addm, w3, as3, ad3, b3)
    xr_ref[...] = _gat(xr, addm, w4, as4, ad4, b4)


def kernel(x, edge_index, W0, as0, ad0, b0, W1, as1, ad1, b1, W2, as2, ad2, b2,
           W3, as3, ad3, b3, W4, as4, ad4, b4):
    n, in_ch = x.shape
    mid = W1.shape[1]
    vmem = pl.BlockSpec(memory_space=pltpu.MemorySpace.VMEM)
    recon, xr, z = pl.pallas_call(
        _body,
        in_specs=[vmem] * 22,
        out_specs=(vmem, vmem, vmem),
        out_shape=(
            jax.ShapeDtypeStruct((n, n), jnp.float32),
            jax.ShapeDtypeStruct((n, in_ch), jnp.float32),
            jax.ShapeDtypeStruct((n, mid), jnp.float32),
        ),
    )(x, edge_index,
      W0, as0.reshape(1, -1), ad0.reshape(1, -1), b0.reshape(1, -1),
      W1, as1.reshape(1, -1), ad1.reshape(1, -1), b1.reshape(1, -1),
      W2, as2.reshape(1, -1), ad2.reshape(1, -1), b2.reshape(1, -1),
      W3, as3.reshape(1, -1), ad3.reshape(1, -1), b3.reshape(1, -1),
      W4, as4.reshape(1, -1), ad4.reshape(1, -1), b4.reshape(1, -1))
    return recon, xr, z
